# baseline (speedup 1.0000x reference)
import functools

import jax
import jax.numpy as jnp
from jax import lax
from jax.experimental import pallas as pl
from jax.experimental.pallas import tpu as pltpu
from jax.experimental.pallas import tpu_sc as plsc

H = 16
L = 2048
NLANE = 16
NCHUNK = L // NLANE
BLK = 8
NBUF = 4

_INFO = plsc.get_sparse_core_info()
NC = _INFO.num_cores
NS = _INFO.num_subcores
NW = NC * NS
ROWS = H * L
RPW = ROWS // NW
NBLK = RPW // BLK
NITER = NBLK // NBUF
assert NBLK % NBUF == 0

_MESH = plsc.VectorSubcoreMesh(core_axis_name="c", subcore_axis_name="s")


@functools.partial(
    pl.kernel,
    out_type=jax.ShapeDtypeStruct((ROWS, L), jnp.float32),
    mesh=_MESH,
    scratch_types=[
        pltpu.VMEM((2 * L,), jnp.float32),
        pltpu.VMEM((NBUF, BLK, L), jnp.float32),
    ] + [pltpu.SemaphoreType.DMA] * (2 * NBUF),
)
def _urpe_sc(ap_hbm, w_hbm, out_hbm, w2_v, buf_v, *sems):
    wid = lax.axis_index("s") * NC + lax.axis_index("c")
    h = wid // (NW // H)
    base_row = wid * RPW
    j0 = base_row - h * L
    sem_ins = sems[:NBUF]
    sem_outs = sems[NBUF:]

    pltpu.sync_copy(w_hbm.at[h, pl.ds(L, L)], w2_v.at[pl.ds(0, L)])
    pltpu.sync_copy(w_hbm.at[h, pl.ds(L, L)], w2_v.at[pl.ds(L, L)])

    def in_copy(g, b):
        return pltpu.make_async_copy(
            ap_hbm.at[pl.ds(base_row + g * BLK, BLK), :], buf_v.at[b],
            sem_ins[b])

    def out_copy(g, b):
        return pltpu.make_async_copy(
            buf_v.at[b], out_hbm.at[pl.ds(base_row + g * BLK, BLK), :],
            sem_outs[b])

    for b in range(NBUF):
        in_copy(b, b).start()

    def step(i, carry):
        for b in range(NBUF):
            g = i * NBUF + b
            in_copy(g, b).wait()
            obase0 = (L - (j0 + g * BLK))

            @plsc.parallel_loop(0, NCHUNK, unroll=4)
            def chunk(c):
                colbase = c * NLANE
                obase = obase0 + colbase
                for br in range(BLK):
                    bias = w2_v[pl.ds(obase - br, NLANE)]
                    a = buf_v[b, br, pl.ds(colbase, NLANE)]
                    buf_v[b, br, pl.ds(colbase, NLANE)] = a * bias

            out_copy(g, b).start()
            bnxt = (b + 2) % NBUF

            @pl.when(jnp.logical_and(g >= 2, g + 2 < NBLK))
            def _():
                out_copy(g - 2, bnxt).wait()
                in_copy(g + 2, bnxt).start()

        return carry

    lax.fori_loop(0, NITER, step, 0)
    for g in range(NBLK - NBUF, NBLK):
        out_copy(g, g % NBUF).wait()


def kernel(attention_probs, urpe_weight_):
    B, Hh, Lq, Lk = attention_probs.shape
    ap2 = attention_probs.reshape(Hh * Lq, Lk)
    out2 = _urpe_sc(ap2, urpe_weight_)
    return out2.reshape(B, Hh, Lq, Lk)


# speedup vs baseline: 1.0793x; 1.0793x over previous
import functools

import jax
import jax.numpy as jnp
from jax import lax
from jax.experimental import pallas as pl
from jax.experimental.pallas import tpu as pltpu
from jax.experimental.pallas import tpu_sc as plsc

H = 16
L = 2048
NLANE = 16
NCHUNK = L // NLANE
BLK = 4
NBUF = 4

_INFO = plsc.get_sparse_core_info()
NC = _INFO.num_cores
NS = _INFO.num_subcores
NW = NC * NS
ROWS = H * L
RPW = ROWS // NW
NBLK = RPW // BLK
NITER = NBLK // NBUF
assert NBLK % NBUF == 0

_MESH = plsc.VectorSubcoreMesh(core_axis_name="c", subcore_axis_name="s")


@functools.partial(
    pl.kernel,
    out_type=jax.ShapeDtypeStruct((ROWS, L), jnp.float32),
    mesh=_MESH,
    scratch_types=[
        pltpu.VMEM((2 * L,), jnp.float32),
        pltpu.VMEM((NBUF, BLK, L), jnp.float32),
        pltpu.VMEM((NBUF, BLK, L), jnp.float32),
    ] + [pltpu.SemaphoreType.DMA] * (2 * NBUF),
)
def _urpe_sc(ap_hbm, w_hbm, out_hbm, w2_v, in_v, out_v, *sems):
    wid = lax.axis_index("s") * NC + lax.axis_index("c")
    h = wid // (NW // H)
    base_row = wid * RPW
    j0 = base_row - h * L
    sem_ins = sems[:NBUF]
    sem_outs = sems[NBUF:]

    pltpu.sync_copy(w_hbm.at[h, pl.ds(L, L)], w2_v.at[pl.ds(0, L)])
    pltpu.sync_copy(w_hbm.at[h, pl.ds(L, L)], w2_v.at[pl.ds(L, L)])

    def in_copy(g, b):
        return pltpu.make_async_copy(
            ap_hbm.at[pl.ds(base_row + g * BLK, BLK), :], in_v.at[b],
            sem_ins[b])

    def out_copy(g, b):
        return pltpu.make_async_copy(
            out_v.at[b], out_hbm.at[pl.ds(base_row + g * BLK, BLK), :],
            sem_outs[b])

    for b in range(NBUF):
        in_copy(b, b).start()

    def step(i, carry):
        for b in range(NBUF):
            g = i * NBUF + b
            in_copy(g, b).wait()

            @pl.when(i > 0)
            def _():
                out_copy(g - NBUF, b).wait()

            obase0 = (L - (j0 + g * BLK))

            @plsc.parallel_loop(0, NCHUNK, unroll=4)
            def chunk(c):
                colbase = c * NLANE
                obase = obase0 + colbase
                for br in range(BLK):
                    bias = w2_v[pl.ds(obase - br, NLANE)]
                    a = in_v[b, br, pl.ds(colbase, NLANE)]
                    out_v[b, br, pl.ds(colbase, NLANE)] = a * bias

            out_copy(g, b).start()

            @pl.when(i < NITER - 1)
            def _():
                in_copy(g + NBUF, b).start()
        return carry

    lax.fori_loop(0, NITER, step, 0)
    for b in range(NBUF):
        out_copy((NITER - 1) * NBUF + b, b).wait()


def kernel(attention_probs, urpe_weight_):
    B, Hh, Lq, Lk = attention_probs.shape
    ap2 = attention_probs.reshape(Hh * Lq, Lk)
    out2 = _urpe_sc(ap2, urpe_weight_)
    return out2.reshape(B, Hh, Lq, Lk)
